# Initial kernel scaffold; baseline (speedup 1.0000x reference)
import jax
import jax.numpy as jnp
from jax.experimental import pallas as pl


def kernel(query_features, gallery_features):
    raise NotImplementedError("write your pallas kernel here")


# speedup vs baseline: 14.8335x; 14.8335x over previous
import functools

import jax
import jax.numpy as jnp
import numpy as np
from jax.experimental import pallas as pl

TOPK = 128
NQ = 1024
NG = 8192
BQ = 256
INT_MIN = np.int32(-2147483648)
MASK31 = np.int32(2147483647)


def _entropy_kernel(q_ref, g_ref, out_ref):
    g = g_ref[...]
    gn = g * jax.lax.rsqrt(jnp.sum(g * g, axis=1, keepdims=True))
    q = q_ref[...]
    qn = q * jax.lax.rsqrt(jnp.sum(q * q, axis=1, keepdims=True))

    sim = jax.lax.dot_general(
        qn, gn, (((1,), (1,)), ((), ())), preferred_element_type=jnp.float32
    )

    bits = jax.lax.bitcast_convert_type(sim, jnp.int32)
    keys = bits ^ ((bits >> 31) & MASK31)

    cnt0 = jnp.sum((keys >= 0).astype(jnp.int32), axis=1, keepdims=True)
    t = jnp.where(cnt0 >= TOPK, jnp.int32(0), INT_MIN)

    def body(i, t):
        bit = jnp.int32(1) << (jnp.int32(30) - i)
        cand = t | bit
        cnt = jnp.sum((keys >= cand).astype(jnp.int32), axis=1, keepdims=True)
        return jnp.where(cnt >= TOPK, cand, t)

    t = jax.lax.fori_loop(0, 31, body, t)

    thr = jax.lax.bitcast_convert_type(t ^ ((t >> 31) & MASK31), jnp.float32)

    gt = keys > t
    cnt_gt = jnp.sum(gt.astype(jnp.float32), axis=1, keepdims=True)
    extra = jnp.float32(TOPK) - cnt_gt

    m = jnp.max(sim, axis=1, keepdims=True)
    e = jnp.where(gt, jnp.exp(sim - m), 0.0)
    et = jnp.exp(thr - m)
    s1 = jnp.sum(e, axis=1, keepdims=True) + extra * et
    s2 = jnp.sum(sim * e, axis=1, keepdims=True) + extra * thr * et
    h = m + jnp.log(s1) - s2 / s1

    @pl.when(pl.program_id(0) == 0)
    def _init():
        out_ref[...] = jnp.zeros_like(out_ref)

    out_ref[...] += jnp.sum(h).reshape(1, 1) * (1.0 / NQ)


@jax.jit
def kernel(query_features, gallery_features):
    out = pl.pallas_call(
        _entropy_kernel,
        grid=(NQ // BQ,),
        in_specs=[
            pl.BlockSpec((BQ, 32), lambda i: (i, 0)),
            pl.BlockSpec((NG, 32), lambda i: (0, 0)),
        ],
        out_specs=pl.BlockSpec((1, 1), lambda i: (0, 0)),
        out_shape=jax.ShapeDtypeStruct((1, 1), jnp.float32),
    )(query_features, gallery_features)
    return out[0, 0]


# speedup vs baseline: 28.5923x; 1.9275x over previous
import jax
import jax.numpy as jnp
import numpy as np
from jax.experimental import pallas as pl

TOPK = 128
NQ = 1024
NG = 8192
BQ = 256
INT_MIN = np.int32(-2147483648)
MASK31 = np.int32(2147483647)
COARSE_BITS = 15


def _key_to_float(t):
    return jax.lax.bitcast_convert_type(t ^ ((t >> 31) & MASK31), jnp.float32)


def _entropy_kernel(q_ref, g_ref, out_ref):
    g = g_ref[...]
    gn = g * jax.lax.rsqrt(jnp.sum(g * g, axis=1, keepdims=True))
    q = q_ref[...]
    qn = q * jax.lax.rsqrt(jnp.sum(q * q, axis=1, keepdims=True))

    sim = jax.lax.dot_general(
        qn, gn, (((1,), (1,)), ((), ())), preferred_element_type=jnp.float32
    )

    kf = jnp.float32(TOPK)

    def count_ge(thr):
        m = (sim >= thr).astype(jnp.float32)
        return jnp.sum(m, axis=1, keepdims=True)

    cnt0 = count_ge(jnp.float32(0.0))
    t = jnp.where(cnt0 >= kf, jnp.int32(0), INT_MIN)

    def body(i, t):
        bit = jnp.int32(1) << (jnp.int32(30) - i)
        cand = t | bit
        cnt = count_ge(_key_to_float(cand))
        return jnp.where(cnt >= kf, cand, t)

    t = jax.lax.fori_loop(0, COARSE_BITS, body, t, unroll=True)
    thr = _key_to_float(t)

    d = sim - thr
    gt = d > 0.0
    gtf = gt.astype(jnp.float32)
    cnt_gt = jnp.sum(gtf, axis=1, keepdims=True)
    e = jnp.where(gt, jnp.exp(d), 0.0)
    extra = kf - cnt_gt
    s1 = jnp.sum(e, axis=1, keepdims=True) + extra
    s2 = jnp.sum(d * e, axis=1, keepdims=True)
    h = jnp.log(s1) - s2 / s1

    @pl.when(pl.program_id(0) == 0)
    def _init():
        out_ref[...] = jnp.zeros_like(out_ref)

    out_ref[...] += jnp.sum(h).reshape(1, 1) * (1.0 / NQ)


@jax.jit
def kernel(query_features, gallery_features):
    out = pl.pallas_call(
        _entropy_kernel,
        grid=(NQ // BQ,),
        in_specs=[
            pl.BlockSpec((BQ, 32), lambda i: (i, 0)),
            pl.BlockSpec((NG, 32), lambda i: (0, 0)),
        ],
        out_specs=pl.BlockSpec((1, 1), lambda i: (0, 0)),
        out_shape=jax.ShapeDtypeStruct((1, 1), jnp.float32),
    )(query_features, gallery_features)
    return out[0, 0]
